# Initial kernel scaffold; baseline (speedup 1.0000x reference)
import jax
import jax.numpy as jnp
from jax.experimental import pallas as pl


def kernel(top_indices, distance_emb):
    raise NotImplementedError("write your pallas kernel here")


# speedup vs baseline: 1.2153x; 1.2153x over previous
import functools

import jax
import jax.numpy as jnp
from jax import lax
from jax.experimental import pallas as pl
from jax.experimental.pallas import tpu as pltpu
from jax.experimental.pallas import tpu_sc as plsc

N_WORDS = 8192
TOP_K = 50
EMB = 64
LANES = 16

NC = 2
NS = 16
NW = NC * NS

B = N_WORDS * TOP_K
PER_W = B // NW
CHUNK = 512
NCH = PER_W // CHUNK
IDX_W = 128
SUB = CHUNK // IDX_W
VECS = CHUNK // LANES

_THRESH = (2, 3, 4, 5, 8, 16, 32, 64)


def _sc_body(ti_hbm, rid_hbm, emb_hbm, out_hbm, ti_v, rid_v, bkt_v, rows_v, sem):
    wid = lax.axis_index("s") * NC + lax.axis_index("c")
    base = wid * PER_W

    def chunk_body(c, carry):
        cbase = base + c * CHUNK
        pltpu.sync_copy(ti_hbm.at[pl.ds(cbase, CHUNK)], ti_v)
        pltpu.sync_copy(rid_hbm.at[pl.ds(cbase, CHUNK)], rid_v)
        for v in range(VECS):
            t = ti_v[pl.ds(v * LANES, LANES)]
            row = rid_v[pl.ds(v * LANES, LANES)]
            d = jnp.maximum(row - t, 1)
            bits = lax.bitcast_convert_type(d.astype(jnp.float32), jnp.int32)
            e = (bits >> 23) - 127
            bkt = jnp.minimum(d, 5) - 1 + jnp.clip(e - 2, 0, 4)
            bkt_v[v // 8, pl.ds((v % 8) * LANES, LANES)] = bkt
        copies = [
            pltpu.async_copy(
                emb_hbm.at[bkt_v.at[g]],
                rows_v.at[pl.ds(g * IDX_W, IDX_W)],
                sem,
            )
            for g in range(SUB)
        ]
        for cp in copies:
            cp.wait()
        pltpu.sync_copy(rows_v, out_hbm.at[pl.ds(cbase, CHUNK)])
        return carry

    lax.fori_loop(0, NCH, chunk_body, 0)


@jax.jit
def kernel(top_indices, distance_emb):
    mesh = plsc.VectorSubcoreMesh(core_axis_name="c", subcore_axis_name="s")
    run = functools.partial(
        pl.kernel,
        mesh=mesh,
        out_type=jax.ShapeDtypeStruct((B, EMB), jnp.float32),
        scratch_types=[
            pltpu.VMEM((CHUNK,), jnp.int32),
            pltpu.VMEM((CHUNK,), jnp.int32),
            pltpu.VMEM((SUB, IDX_W), jnp.int32),
            pltpu.VMEM((CHUNK, EMB), jnp.float32),
            pltpu.SemaphoreType.DMA,
        ],
        compiler_params=pltpu.CompilerParams(use_tc_tiling_on_sc=False),
    )(_sc_body)
    row_ids = jnp.repeat(jnp.arange(N_WORDS, dtype=jnp.int32), TOP_K)
    flat = run(top_indices.reshape(B).astype(jnp.int32), row_ids, distance_emb)
    return flat.reshape(N_WORDS, TOP_K, EMB)


# speedup vs baseline: 11.1963x; 9.2131x over previous
import functools

import jax
import jax.numpy as jnp
from jax import lax
from jax.experimental import pallas as pl
from jax.experimental.pallas import tpu as pltpu
from jax.experimental.pallas import tpu_sc as plsc

N_WORDS = 8192
TOP_K = 50
EMB = 64
LANES = 16

NC = 2
NS = 16
NW = NC * NS

B = N_WORDS * TOP_K
PER_W = B // NW
CHUNK = 640
NCH = PER_W // CHUNK
VECS = CHUNK // LANES
TAB = 9 * EMB


def _bucket(d):
    bits = lax.bitcast_convert_type(d.astype(jnp.float32), jnp.int32)
    e = (bits >> 23) - 127
    return jnp.minimum(d, 5) - 1 + jnp.clip(e - 2, 0, 4)


def _sc_body(ti_hbm, rid_hbm, emb_hbm, out_hbm,
             ti_v, rid_v, tab_v, rows0, rows1, sem0, sem1):
    wid = lax.axis_index("s") * NC + lax.axis_index("c")
    base = wid * PER_W
    pltpu.sync_copy(ti_hbm.at[pl.ds(base, PER_W)], ti_v)
    pltpu.sync_copy(rid_hbm.at[pl.ds(base, PER_W)], rid_v)
    pltpu.sync_copy(emb_hbm, tab_v)
    lane = lax.iota(jnp.int32, LANES)

    def compute_chunk(c, rows):
        def vbody(v, _):
            off = c * CHUNK + v * LANES
            t = ti_v[pl.ds(off, LANES)]
            row = rid_v[pl.ds(off, LANES)]
            bkt64 = _bucket(jnp.maximum(row - t, 1)) * EMB
            for k in range(LANES):
                b64 = jnp.take(bkt64, jnp.full((LANES,), k, jnp.int32))
                sbase = (v * LANES + k) * EMB
                for j in range(EMB // LANES):
                    vals = plsc.load_gather(tab_v, [b64 + (j * LANES + lane)])
                    rows[pl.ds(sbase + j * LANES, LANES)] = vals
            return 0
        lax.fori_loop(0, VECS, vbody, 0)

    def out_copy(c, rows, sem):
        return pltpu.make_async_copy(
            rows, out_hbm.at[pl.ds((base + c * CHUNK) * EMB, CHUNK * EMB)], sem)

    def outer(cc, _):
        for b, (rows, sem) in enumerate(((rows0, sem0), (rows1, sem1))):
            c = cc * 2 + b

            @pl.when(cc > 0)
            def _wait():
                out_copy(c - 2, rows, sem).wait()

            compute_chunk(c, rows)
            out_copy(c, rows, sem).start()
        return 0

    lax.fori_loop(0, NCH // 2, outer, 0)
    out_copy(NCH - 2, rows0, sem0).wait()
    out_copy(NCH - 1, rows1, sem1).wait()


@jax.jit
def kernel(top_indices, distance_emb):
    mesh = plsc.VectorSubcoreMesh(core_axis_name="c", subcore_axis_name="s")
    run = functools.partial(
        pl.kernel,
        mesh=mesh,
        out_type=jax.ShapeDtypeStruct((B * EMB,), jnp.float32),
        scratch_types=[
            pltpu.VMEM((PER_W,), jnp.int32),
            pltpu.VMEM((PER_W,), jnp.int32),
            pltpu.VMEM((TAB,), jnp.float32),
            pltpu.VMEM((CHUNK * EMB,), jnp.float32),
            pltpu.VMEM((CHUNK * EMB,), jnp.float32),
            pltpu.SemaphoreType.DMA,
            pltpu.SemaphoreType.DMA,
        ],
        compiler_params=pltpu.CompilerParams(
            use_tc_tiling_on_sc=False, needs_layout_passes=False),
    )(_sc_body)
    row_ids = jnp.repeat(jnp.arange(N_WORDS, dtype=jnp.int32), TOP_K)
    flat = run(top_indices.reshape(B).astype(jnp.int32), row_ids,
               distance_emb.reshape(TAB))
    return flat.reshape(N_WORDS, TOP_K, EMB)


# speedup vs baseline: 13.2582x; 1.1842x over previous
import functools

import jax
import jax.numpy as jnp
from jax import lax
from jax.experimental import pallas as pl
from jax.experimental.pallas import tpu as pltpu
from jax.experimental.pallas import tpu_sc as plsc

N_WORDS = 8192
TOP_K = 50
EMB = 64
LANES = 16

NC = 2
NS = 16
NW = NC * NS

B = N_WORDS * TOP_K
PER_W = B // NW
CHUNK = 640
NCH = PER_W // CHUNK
VECS = CHUNK // LANES
TAB = 9 * EMB


def _bucket(d):
    bits = lax.bitcast_convert_type(d.astype(jnp.float32), jnp.int32)
    e = (bits >> 23) - 127
    return jnp.minimum(d, 5) - 1 + jnp.clip(e - 2, 0, 4)


def _sc_body(ti_hbm, rid_hbm, emb_hbm, out_hbm,
             ti_v, rid_v, tab_v, rows0, rows1, sem0, sem1):
    wid = lax.axis_index("s") * NC + lax.axis_index("c")
    base = wid * PER_W
    pltpu.sync_copy(ti_hbm.at[pl.ds(base, PER_W)], ti_v)
    pltpu.sync_copy(rid_hbm.at[pl.ds(base, PER_W)], rid_v)
    pltpu.sync_copy(emb_hbm, tab_v)
    lane = lax.iota(jnp.int32, LANES)

    def compute_chunk(c, rows):
        @plsc.parallel_loop(0, VECS, unroll=2)
        def vbody(v):
            off = c * CHUNK + v * LANES
            t = ti_v[pl.ds(off, LANES)]
            row = rid_v[pl.ds(off, LANES)]
            bkt64 = _bucket(jnp.maximum(row - t, 1)) * EMB
            for k in range(LANES):
                b64 = jnp.take(bkt64, jnp.full((LANES,), k, jnp.int32))
                sbase = (v * LANES + k) * EMB
                for j in range(EMB // LANES):
                    vals = plsc.load_gather(tab_v, [b64 + (j * LANES + lane)])
                    rows[pl.ds(sbase + j * LANES, LANES)] = vals

    def out_copy(c, rows, sem):
        return pltpu.make_async_copy(
            rows, out_hbm.at[pl.ds((base + c * CHUNK) * EMB, CHUNK * EMB)], sem)

    def outer(cc, _):
        for b, (rows, sem) in enumerate(((rows0, sem0), (rows1, sem1))):
            c = cc * 2 + b

            @pl.when(cc > 0)
            def _wait():
                out_copy(c - 2, rows, sem).wait()

            compute_chunk(c, rows)
            out_copy(c, rows, sem).start()
        return 0

    lax.fori_loop(0, NCH // 2, outer, 0)
    out_copy(NCH - 2, rows0, sem0).wait()
    out_copy(NCH - 1, rows1, sem1).wait()


@jax.jit
def kernel(top_indices, distance_emb):
    mesh = plsc.VectorSubcoreMesh(core_axis_name="c", subcore_axis_name="s")
    run = functools.partial(
        pl.kernel,
        mesh=mesh,
        out_type=jax.ShapeDtypeStruct((B * EMB,), jnp.float32),
        scratch_types=[
            pltpu.VMEM((PER_W,), jnp.int32),
            pltpu.VMEM((PER_W,), jnp.int32),
            pltpu.VMEM((TAB,), jnp.float32),
            pltpu.VMEM((CHUNK * EMB,), jnp.float32),
            pltpu.VMEM((CHUNK * EMB,), jnp.float32),
            pltpu.SemaphoreType.DMA,
            pltpu.SemaphoreType.DMA,
        ],
        compiler_params=pltpu.CompilerParams(
            use_tc_tiling_on_sc=False, needs_layout_passes=False),
    )(_sc_body)
    row_ids = jnp.repeat(jnp.arange(N_WORDS, dtype=jnp.int32), TOP_K)
    flat = run(top_indices.reshape(B).astype(jnp.int32), row_ids,
               distance_emb.reshape(TAB))
    return flat.reshape(N_WORDS, TOP_K, EMB)
